# baseline (speedup 1.0000x reference)
import functools

import jax
import jax.numpy as jnp
from jax import lax
from jax.experimental import pallas as pl
from jax.experimental.pallas import tpu as pltpu
from jax.experimental.pallas import tpu_sc as plsc

VOCAB = 100000
D = 768
B = 4
T = 2048

_INFO = plsc.get_sparse_core_info()
NC, NS, L = _INFO.num_cores, _INFO.num_subcores, _INFO.num_lanes
NW = NC * NS
T_PER_W = T // NW
CHUNK = 16
GROUPS = T_PER_W // CHUNK


def _body(ids_hbm, pos_hbm, tok_hbm, out_hbm,
          idx_v, rows_v, pos_v, sem_i,
          sem_g0, sem_g1, sem_o0, sem_o1):
    wid = lax.axis_index("c") * NS + lax.axis_index("s")
    tbase = wid * T_PER_W
    sem_g = (sem_g0, sem_g1)
    sem_o = (sem_o0, sem_o1)

    idx_cps = []
    for grp in range(GROUPS):
        for b in range(B):
            idx_cps.append(pltpu.async_copy(
                ids_hbm.at[b, pl.ds(tbase + grp * CHUNK, CHUNK)],
                idx_v.at[grp * B + b], sem_i))
    for cp in idx_cps:
        cp.wait()

    def start_group(grp):
        h = grp % 2
        cps = [pltpu.async_copy(pos_hbm.at[pl.ds(tbase + grp * CHUNK, CHUNK)],
                                pos_v.at[h], sem_g[h])]
        for b in range(B):
            cps.append(pltpu.async_copy(tok_hbm.at[idx_v.at[grp * B + b]],
                                        rows_v.at[h * B + b], sem_g[h]))
        return cps

    pend = {0: start_group(0)}
    outs = {}
    for grp in range(GROUPS):
        h = grp % 2
        if grp + 1 < GROUPS:
            if grp - 1 >= 0:
                for cp in outs[grp - 1]:
                    cp.wait()
            pend[grp + 1] = start_group(grp + 1)
        for cp in pend[grp]:
            cp.wait()

        outs[grp] = []
        HALF = CHUNK // 2
        for half in range(2):
            @plsc.parallel_loop(half * HALF, (half + 1) * HALF, step=1,
                                unroll=1)
            def row_step(r, h=h):
                for g in range(D // L):
                    sl = pl.ds(g * L, L)
                    v = pos_v[h, r, sl]
                    for b in range(B):
                        plsc.addupdate(rows_v.at[h * B + b, r, sl], v)

            outs[grp] += [pltpu.async_copy(
                rows_v.at[h * B + b, pl.ds(half * HALF, HALF)],
                out_hbm.at[b, pl.ds(tbase + grp * CHUNK + half * HALF, HALF)],
                sem_o[h]) for b in range(B)]

    for grp in (GROUPS - 2, GROUPS - 1):
        for cp in outs[grp]:
            cp.wait()


@jax.jit
def _embed(ids, tok_emb, pos_emb):
    mesh = plsc.VectorSubcoreMesh(core_axis_name="c", subcore_axis_name="s")
    k = functools.partial(
        pl.kernel,
        mesh=mesh,
        out_type=jax.ShapeDtypeStruct((B, T, D), jnp.float32),
        scratch_types=[
            pltpu.VMEM((GROUPS * B, CHUNK), jnp.int32),
            pltpu.VMEM((2 * B, CHUNK, D), jnp.float32),
            pltpu.VMEM((2, CHUNK, D), jnp.float32),
            pltpu.SemaphoreType.DMA,
            pltpu.SemaphoreType.DMA,
            pltpu.SemaphoreType.DMA,
            pltpu.SemaphoreType.DMA,
            pltpu.SemaphoreType.DMA,
        ],
    )(_body)
    return k(ids, pos_emb, tok_emb)


def kernel(input_ids, tok_emb, pos_emb):
    return _embed(input_ids.astype(jnp.int32), tok_emb, pos_emb)


# speedup vs baseline: 1.0302x; 1.0302x over previous
import functools

import jax
import jax.numpy as jnp
from jax import lax
from jax.experimental import pallas as pl
from jax.experimental.pallas import tpu as pltpu
from jax.experimental.pallas import tpu_sc as plsc

VOCAB = 100000
D = 768
B = 4
T = 2048

_INFO = plsc.get_sparse_core_info()
NC, NS, L = _INFO.num_cores, _INFO.num_subcores, _INFO.num_lanes
NW = NC * NS
T_PER_W = T // NW
CHUNK = 16
GROUPS = T_PER_W // CHUNK


def _body(ids_hbm, pos_hbm, tok_hbm, out_hbm,
          idx_v, rows_v, pos_v, sem_i,
          sem_p0, sem_p1, sem_g0, sem_g1,
          sem_o0, sem_o1, sem_o2, sem_o3, sem_o4, sem_o5, sem_o6, sem_o7):
    wid = lax.axis_index("c") * NS + lax.axis_index("s")
    tbase = wid * T_PER_W
    sem_p = (sem_p0, sem_p1)
    sem_g = (sem_g0, sem_g1)
    sem_o = ((sem_o0, sem_o1, sem_o2, sem_o3),
             (sem_o4, sem_o5, sem_o6, sem_o7))

    idx_cps = []
    for grp in range(GROUPS):
        for b in range(B):
            idx_cps.append(pltpu.async_copy(
                ids_hbm.at[b, pl.ds(tbase + grp * CHUNK, CHUNK)],
                idx_v.at[grp * B + b], sem_i))
    for cp in idx_cps:
        cp.wait()

    def start_group(grp, prev_outs):
        h = grp % 2
        cps = [pltpu.async_copy(pos_hbm.at[pl.ds(tbase + grp * CHUNK, CHUNK)],
                                pos_v.at[h], sem_p[h])]
        for b in range(B):
            if prev_outs is not None:
                prev_outs[b].wait()
            cps.append(pltpu.async_copy(tok_hbm.at[idx_v.at[grp * B + b]],
                                        rows_v.at[h * B + b], sem_g[h]))
        return cps

    pend = {0: start_group(0, None)}
    outs = {}
    for grp in range(GROUPS):
        h = grp % 2
        if grp + 1 < GROUPS:
            pend[grp + 1] = start_group(grp + 1, outs.get(grp - 1))
        for cp in pend[grp]:
            cp.wait()

        @plsc.parallel_loop(0, CHUNK, step=1, unroll=1)
        def row_step(r, h=h):
            for g in range(D // L):
                sl = pl.ds(g * L, L)
                v = pos_v[h, r, sl]
                for b in range(B):
                    plsc.addupdate(rows_v.at[h * B + b, r, sl], v)

        outs[grp] = [pltpu.async_copy(
            rows_v.at[h * B + b],
            out_hbm.at[b, pl.ds(tbase + grp * CHUNK, CHUNK)],
            sem_o[h][b]) for b in range(B)]

    for grp in (GROUPS - 2, GROUPS - 1):
        for cp in outs[grp]:
            cp.wait()


@jax.jit
def _embed(ids, tok_emb, pos_emb):
    mesh = plsc.VectorSubcoreMesh(core_axis_name="c", subcore_axis_name="s")
    k = functools.partial(
        pl.kernel,
        mesh=mesh,
        out_type=jax.ShapeDtypeStruct((B, T, D), jnp.float32),
        scratch_types=[
            pltpu.VMEM((GROUPS * B, CHUNK), jnp.int32),
            pltpu.VMEM((2 * B, CHUNK, D), jnp.float32),
            pltpu.VMEM((2, CHUNK, D), jnp.float32),
        ] + [pltpu.SemaphoreType.DMA] * 13,
    )(_body)
    return k(ids, pos_emb, tok_emb)


def kernel(input_ids, tok_emb, pos_emb):
    return _embed(input_ids.astype(jnp.int32), tok_emb, pos_emb)
